# baseline (speedup 1.0000x reference)
import functools

import jax
import jax.numpy as jnp
from jax import lax
from jax.experimental import pallas as pl
from jax.experimental.pallas import tpu as pltpu
from jax.experimental.pallas import tpu_sc as plsc

B = 128
V = 100000
TOPP = 0.95
PEN_W = 10.0
SIL = (1049, 127, 1880, 1492, 972, 1031, 395, 2029, 581, 175, 1926, 407, 1316)

L = 16
NW = 32
ROWS_PER_W = B // NW

NB1 = 512
NB2 = 1024
LO = -26.0
W1 = 32.0 / NB1
W2 = W1 / NB2
INV_W1 = 1.0 / W1
INV_W2 = 1.0 / W2

GCH = 4000
NGC = V // GCH
VREGS_ROW = V // L
VREGS_GCH = GCH // L

BIG = 3.0e38
BIGI = 2**30


def _f1_of(xv):
    t = ((xv - LO) * INV_W1).astype(jnp.int32)
    return jnp.minimum(jnp.maximum(t, 0), NB1 - 1)


def _f2_of(xv, lo2):
    t = ((xv - lo2) * INV_W2).astype(jnp.int32)
    return jnp.minimum(jnp.maximum(t, 0), NB2 - 1)


def _body(x_hbm, g_hbm, out_hbm, xrow, hflat, gbuf0, gbuf1, tref,
          cand_x, cand_g, cand_i, resbuf, sem0, sem1):
    wid = lax.axis_index("s") * 2 + lax.axis_index("c")
    iota = lax.iota(jnp.int32, L)
    fzero = jnp.zeros((L,), jnp.float32)
    sil_mask = iota < len(SIL)
    sil_idx = jnp.zeros((L,), jnp.int32)
    for k, s in enumerate(SIL):
        sil_idx = jnp.where(iota == k, s, sil_idx)

    def zero_h(nwords):
        @plsc.parallel_loop(0, nwords // L, unroll=8)
        def _(i):
            hflat[pl.ds(i * L, L)] = fzero

    def bin_totals(nb):
        @plsc.parallel_loop(0, nb // L, unroll=2)
        def _(cb):
            acc = fzero
            for l in range(L):
                acc = acc + hflat[pl.ds(l * nb + cb * L, L)]
            tref[pl.ds(cb * L, L)] = acc

    def find_cross(nb, a0, p_c):
        def st(t, carry):
            acc, found, beta, a = carry
            cb = nb // L - 1 - t
            tv = tref[pl.ds(cb * L, L)]
            rv = lax.rev(tv, (0,))
            cs = plsc.cumsum(rv)
            m = (acc + cs) > p_c
            anyc = jnp.any(m)
            k = jnp.min(jnp.where(m, iota, L))
            ak = acc + jnp.sum(jnp.where(iota < k, rv, 0.0))
            betak = cb * L + (L - 1) - k
            use = anyc & jnp.logical_not(found)
            beta = jnp.where(use, betak, beta)
            a = jnp.where(use, ak, a)
            return (acc + jnp.sum(tv), found | anyc, beta, a)
        _, _, beta, a = lax.fori_loop(
            0, nb // L, st, (a0, False, jnp.int32(0), jnp.float32(0.0)))
        return beta, a

    def row_body(j, resvec):
        row = wid * ROWS_PER_W + j
        pltpu.sync_copy(x_hbm.at[pl.ds(row * V, V)], xrow)
        plsc.addupdate_scatter(xrow, [sil_idx],
                               jnp.full((L,), -PEN_W, jnp.float32),
                               mask=sil_mask)
        zero_h(NB1 * L)

        @plsc.parallel_loop(0, VREGS_ROW, unroll=4)
        def _(i):
            xv = xrow[pl.ds(i * L, L)]
            ev = jnp.exp(xv)
            addr = iota * NB1 + _f1_of(xv)
            plsc.addupdate_scatter(hflat, [addr], ev)
        bin_totals(NB1)

        def csum(cb, acc):
            return acc + tref[pl.ds(cb * L, L)]
        c_total = jnp.sum(lax.fori_loop(0, NB1 // L, csum, fzero))
        p_c = jnp.float32(TOPP) * c_total
        beta1, a1 = find_cross(NB1, jnp.float32(0.0), p_c)
        lo2 = jnp.float32(LO) + beta1.astype(jnp.float32) * jnp.float32(W1)

        zero_h(NB2 * L)

        @plsc.parallel_loop(0, VREGS_ROW, unroll=4)
        def _(i):
            xv = xrow[pl.ds(i * L, L)]
            ev = jnp.exp(xv)
            inb = _f1_of(xv) == beta1
            addr = iota * NB2 + _f2_of(xv, lo2)
            plsc.addupdate_scatter(hflat, [addr], ev, mask=inb)
        bin_totals(NB2)
        beta2, a2 = find_cross(NB2, a1, p_c)

        def chunk_body(cbase, gbuf, carry):
            @plsc.parallel_loop(0, VREGS_GCH, unroll=4, carry=carry)
            def p4(i, car):
                bestv, besti, cnt = car
                xv = xrow[pl.ds(cbase + i * L, L)]
                gv = gbuf[pl.ds(i * L, L)]
                f1 = _f1_of(xv)
                inb = f1 == beta1
                f2 = _f2_of(xv, lo2)
                above = (f1 > beta1) | (inb & (f2 > beta2))
                scv = jnp.where(above, xv + gv, -BIG)
                idxv = iota + (cbase + i * L)
                upd = scv > bestv
                bestv = jnp.where(upd, scv, bestv)
                besti = jnp.where(upd, idxv, besti)
                candm = inb & (f2 == beta2)

                @pl.when(jnp.any(candm))
                def _():
                    pos = jnp.minimum(
                        cnt + plsc.cumsum(candm.astype(jnp.int32)) - 1, 31)
                    plsc.store_scatter(cand_x, [pos], xv, mask=candm)
                    plsc.store_scatter(cand_g, [pos], gv, mask=candm)
                    plsc.store_scatter(cand_i, [pos], idxv, mask=candm)
                cnt = cnt + plsc.all_reduce_population_count(candm)
                return (bestv, besti, cnt)
            return p4

        carry = (jnp.full((L,), -BIG, jnp.float32), jnp.zeros((L,), jnp.int32),
                 jnp.zeros((L,), jnp.int32))
        gbufs = (gbuf0, gbuf1)
        sems = (sem0, sem1)
        gbase = row * V
        desc = pltpu.async_copy(g_hbm.at[pl.ds(gbase, GCH)], gbuf0, sem0)
        for c in range(NGC):
            nxt = None
            if c + 1 < NGC:
                nxt = pltpu.async_copy(
                    g_hbm.at[pl.ds(gbase + (c + 1) * GCH, GCH)],
                    gbufs[(c + 1) % 2], sems[(c + 1) % 2])
            desc.wait()
            carry = chunk_body(c * GCH, gbufs[c % 2], carry)
            desc = nxt
        bestv, besti, cnt = carry

        ma = jnp.max(bestv)
        ia = jnp.min(jnp.where(bestv == ma, besti, BIGI))
        ncand = jnp.max(cnt)

        cx = cand_x[pl.ds(0, L)]
        cg = cand_g[pl.ds(0, L)]
        ci = cand_i[pl.ds(0, L)]
        ce = jnp.exp(cx)
        sg = fzero
        eqle = jnp.zeros((L,), jnp.int32)
        for jj in range(L):
            validj = ncand > jj
            xj = cx[jj]
            ij = ci[jj]
            ej = ce[jj]
            sg = sg + jnp.where(validj & (xj > cx), ej, 0.0)
            eqle = eqle + jnp.where(validj & (xj == cx) & (ij <= ci), 1, 0)
        lane_valid = iota < ncand
        cv = a2 + sg + ce * eqle.astype(jnp.float32)
        ckept = lane_valid & ((cv - ce) <= p_c)
        cscore = jnp.where(ckept, cx + cg, -BIG)
        mc = jnp.max(cscore)
        ic = jnp.min(jnp.where(cscore == mc, ci, BIGI))
        use_c = (mc > ma) | ((mc == ma) & (ic < ia))
        res = jnp.where(use_c, ic, ia)
        return jnp.where(iota == j, res, resvec)

    resvec = lax.fori_loop(0, ROWS_PER_W, row_body, jnp.zeros((L,), jnp.int32))
    resbuf[pl.ds(0, L)] = resvec
    pltpu.sync_copy(resbuf, out_hbm.at[wid])


def _gumbel_const():
    skey = jax.random.fold_in(jax.random.key(0), 1)
    return jax.random.gumbel(skey, (B * V,), jnp.float32)


_G = _gumbel_const()


@functools.cache
def _sc_call():
    mesh = plsc.VectorSubcoreMesh(core_axis_name="c", subcore_axis_name="s",
                                  num_cores=2, num_subcores=16)
    return pl.kernel(
        _body,
        out_type=jax.ShapeDtypeStruct((NW, L), jnp.int32),
        mesh=mesh,
        compiler_params=pltpu.CompilerParams(use_tc_tiling_on_sc=False,
                                             needs_layout_passes=False),
        scratch_types=[
            pltpu.VMEM((V,), jnp.float32),
            pltpu.VMEM((NB2 * L,), jnp.float32),
            pltpu.VMEM((GCH,), jnp.float32),
            pltpu.VMEM((GCH,), jnp.float32),
            pltpu.VMEM((NB2,), jnp.float32),
            pltpu.VMEM((32,), jnp.float32),
            pltpu.VMEM((32,), jnp.float32),
            pltpu.VMEM((32,), jnp.int32),
            pltpu.VMEM((L,), jnp.int32),
            pltpu.SemaphoreType.DMA,
            pltpu.SemaphoreType.DMA,
        ],
    )


def kernel(logits):
    assert logits.shape == (B, V) and logits.dtype == jnp.float32
    staging = _sc_call()(logits.reshape(B * V), _G)
    return staging[:, :ROWS_PER_W].reshape(B, 1).astype(jnp.int32)


# speedup vs baseline: 1.3530x; 1.3530x over previous
import functools

import jax
import jax.numpy as jnp
from jax import lax
from jax.experimental import pallas as pl
from jax.experimental.pallas import tpu as pltpu
from jax.experimental.pallas import tpu_sc as plsc

B = 128
V = 100000
TOPP = 0.95
PEN_W = 10.0
SIL = (1049, 127, 1880, 1492, 972, 1031, 395, 2029, 581, 175, 1926, 407, 1316)

L = 16
NW = 32
ROWS_PER_W = B // NW

NB1 = 512
NB2 = 1024
LO = -26.0
W1 = 32.0 / NB1
W2 = W1 / NB2
INV_W1 = 1.0 / W1
INV_W2 = 1.0 / W2

GCH = 4000
NGC = V // GCH
VREGS_ROW = V // L
VREGS_GCH = GCH // L

BIG = 3.0e38
BIGI = 2**30


def _f1_of(xv):
    t = ((xv - LO) * INV_W1).astype(jnp.int32)
    return jnp.minimum(jnp.maximum(t, 0), NB1 - 1)


def _f2_of(xv, lo2):
    t = ((xv - lo2) * INV_W2).astype(jnp.int32)
    return jnp.minimum(jnp.maximum(t, 0), NB2 - 1)


def _body(x_hbm, g_hbm, out_hbm, xrow, hflat, gbuf0, gbuf1, tref,
          cand_x, cand_g, cand_i, resbuf, sem0, sem1):
    wid = lax.axis_index("s") * 2 + lax.axis_index("c")
    iota = lax.iota(jnp.int32, L)
    fzero = jnp.zeros((L,), jnp.float32)
    sil_mask = iota < len(SIL)
    sil_idx = jnp.zeros((L,), jnp.int32)
    for k, s in enumerate(SIL):
        sil_idx = jnp.where(iota == k, s, sil_idx)

    def zero_h(nwords):
        @plsc.parallel_loop(0, nwords // L, unroll=8)
        def _(i):
            hflat[pl.ds(i * L, L)] = fzero

    def bin_totals(nb):
        @plsc.parallel_loop(0, nb // L, unroll=2)
        def _(cb):
            acc = fzero
            for k in range(L):
                s = jnp.sum(hflat[pl.ds((cb * L + k) * L, L)])
                acc = jnp.where(iota == k, s, acc)
            tref[pl.ds(cb * L, L)] = acc

    def find_cross(nb, a0, p_c):
        def st(t, carry):
            acc, found, beta, a = carry
            cb = nb // L - 1 - t
            tv = tref[pl.ds(cb * L, L)]
            rv = lax.rev(tv, (0,))
            cs = plsc.cumsum(rv)
            m = (acc + cs) > p_c
            anyc = jnp.any(m)
            k = jnp.min(jnp.where(m, iota, L))
            ak = acc + jnp.sum(jnp.where(iota < k, rv, 0.0))
            betak = cb * L + (L - 1) - k
            use = anyc & jnp.logical_not(found)
            beta = jnp.where(use, betak, beta)
            a = jnp.where(use, ak, a)
            return (acc + jnp.sum(tv), found | anyc, beta, a)
        _, _, beta, a = lax.fori_loop(
            0, nb // L, st, (a0, False, jnp.int32(0), jnp.float32(0.0)))
        return beta, a

    def row_body(j, resvec):
        row = wid * ROWS_PER_W + j
        pltpu.sync_copy(x_hbm.at[pl.ds(row * V, V)], xrow)
        plsc.addupdate_scatter(xrow, [sil_idx],
                               jnp.full((L,), -PEN_W, jnp.float32),
                               mask=sil_mask)
        zero_h(NB1 * L)

        @plsc.parallel_loop(0, VREGS_ROW, unroll=8)
        def _(i):
            xv = xrow[pl.ds(i * L, L)]
            ev = jnp.exp(xv)
            addr = _f1_of(xv) * L + iota
            plsc.addupdate_scatter(hflat, [addr], ev)
        bin_totals(NB1)

        def csum(cb, acc):
            return acc + tref[pl.ds(cb * L, L)]
        c_total = jnp.sum(lax.fori_loop(0, NB1 // L, csum, fzero))
        p_c = jnp.float32(TOPP) * c_total
        beta1, a1 = find_cross(NB1, jnp.float32(0.0), p_c)
        lo2 = jnp.float32(LO) + beta1.astype(jnp.float32) * jnp.float32(W1)

        zero_h(NB2 * L)

        @plsc.parallel_loop(0, VREGS_ROW, unroll=8)
        def _(i):
            xv = xrow[pl.ds(i * L, L)]
            ev = jnp.exp(xv)
            inb = _f1_of(xv) == beta1
            addr = _f2_of(xv, lo2) * L + iota
            plsc.addupdate_scatter(hflat, [addr], ev, mask=inb)
        bin_totals(NB2)
        beta2, a2 = find_cross(NB2, a1, p_c)

        def chunk_body(cbase, gbuf, carry):
            @plsc.parallel_loop(0, VREGS_GCH, unroll=4, carry=carry)
            def p4(i, car):
                bestv, besti, cnt = car
                xv = xrow[pl.ds(cbase + i * L, L)]
                gv = gbuf[pl.ds(i * L, L)]
                f1 = _f1_of(xv)
                inb = f1 == beta1
                f2 = _f2_of(xv, lo2)
                above = (f1 > beta1) | (inb & (f2 > beta2))
                scv = jnp.where(above, xv + gv, -BIG)
                idxv = iota + (cbase + i * L)
                upd = scv > bestv
                bestv = jnp.where(upd, scv, bestv)
                besti = jnp.where(upd, idxv, besti)
                candm = inb & (f2 == beta2)

                @pl.when(jnp.any(candm))
                def _():
                    pos = jnp.minimum(
                        cnt + plsc.cumsum(candm.astype(jnp.int32)) - 1, 31)
                    plsc.store_scatter(cand_x, [pos], xv, mask=candm)
                    plsc.store_scatter(cand_g, [pos], gv, mask=candm)
                    plsc.store_scatter(cand_i, [pos], idxv, mask=candm)
                cnt = cnt + plsc.all_reduce_population_count(candm)
                return (bestv, besti, cnt)
            return p4

        carry = (jnp.full((L,), -BIG, jnp.float32), jnp.zeros((L,), jnp.int32),
                 jnp.zeros((L,), jnp.int32))
        gbufs = (gbuf0, gbuf1)
        sems = (sem0, sem1)
        gbase = row * V
        desc = pltpu.async_copy(g_hbm.at[pl.ds(gbase, GCH)], gbuf0, sem0)
        for c in range(NGC):
            nxt = None
            if c + 1 < NGC:
                nxt = pltpu.async_copy(
                    g_hbm.at[pl.ds(gbase + (c + 1) * GCH, GCH)],
                    gbufs[(c + 1) % 2], sems[(c + 1) % 2])
            desc.wait()
            carry = chunk_body(c * GCH, gbufs[c % 2], carry)
            desc = nxt
        bestv, besti, cnt = carry

        ma = jnp.max(bestv)
        ia = jnp.min(jnp.where(bestv == ma, besti, BIGI))
        ncand = jnp.max(cnt)

        cx = cand_x[pl.ds(0, L)]
        cg = cand_g[pl.ds(0, L)]
        ci = cand_i[pl.ds(0, L)]
        ce = jnp.exp(cx)
        sg = fzero
        eqle = jnp.zeros((L,), jnp.int32)
        for jj in range(L):
            validj = ncand > jj
            xj = cx[jj]
            ij = ci[jj]
            ej = ce[jj]
            sg = sg + jnp.where(validj & (xj > cx), ej, 0.0)
            eqle = eqle + jnp.where(validj & (xj == cx) & (ij <= ci), 1, 0)
        lane_valid = iota < ncand
        cv = a2 + sg + ce * eqle.astype(jnp.float32)
        ckept = lane_valid & ((cv - ce) <= p_c)
        cscore = jnp.where(ckept, cx + cg, -BIG)
        mc = jnp.max(cscore)
        ic = jnp.min(jnp.where(cscore == mc, ci, BIGI))
        use_c = (mc > ma) | ((mc == ma) & (ic < ia))
        res = jnp.where(use_c, ic, ia)
        return jnp.where(iota == j, res, resvec)

    resvec = lax.fori_loop(0, ROWS_PER_W, row_body, jnp.zeros((L,), jnp.int32))
    resbuf[pl.ds(0, L)] = resvec
    pltpu.sync_copy(resbuf, out_hbm.at[wid])


def _gumbel_const():
    skey = jax.random.fold_in(jax.random.key(0), 1)
    return jax.random.gumbel(skey, (B * V,), jnp.float32)


@functools.cache
def _sc_call():
    mesh = plsc.VectorSubcoreMesh(core_axis_name="c", subcore_axis_name="s",
                                  num_cores=2, num_subcores=16)
    return pl.kernel(
        _body,
        out_type=jax.ShapeDtypeStruct((NW, L), jnp.int32),
        mesh=mesh,
        compiler_params=pltpu.CompilerParams(use_tc_tiling_on_sc=False,
                                             needs_layout_passes=False),
        scratch_types=[
            pltpu.VMEM((V,), jnp.float32),
            pltpu.VMEM((NB2 * L,), jnp.float32),
            pltpu.VMEM((GCH,), jnp.float32),
            pltpu.VMEM((GCH,), jnp.float32),
            pltpu.VMEM((NB2,), jnp.float32),
            pltpu.VMEM((32,), jnp.float32),
            pltpu.VMEM((32,), jnp.float32),
            pltpu.VMEM((32,), jnp.int32),
            pltpu.VMEM((L,), jnp.int32),
            pltpu.SemaphoreType.DMA,
            pltpu.SemaphoreType.DMA,
        ],
    )


def kernel(logits):
    assert logits.shape == (B, V) and logits.dtype == jnp.float32
    staging = _sc_call()(logits.reshape(B * V), _gumbel_const())
    return staging[:, :ROWS_PER_W].reshape(B, 1).astype(jnp.int32)


# speedup vs baseline: 1.7693x; 1.3077x over previous
import functools

import jax
import jax.numpy as jnp
from jax import lax
from jax.experimental import pallas as pl
from jax.experimental.pallas import tpu as pltpu
from jax.experimental.pallas import tpu_sc as plsc

B = 128
V = 100000
TOPP = 0.95
PEN_W = 10.0
SIL = (1049, 127, 1880, 1492, 972, 1031, 395, 2029, 581, 175, 1926, 407, 1316)

L = 16
NW = 32
ROWS_PER_W = B // NW

NB1 = 512
NB2 = 512
LO = -26.0
W1 = 32.0 / NB1
W2 = W1 / NB2
INV_W1 = 1.0 / W1
INV_W2 = 1.0 / W2

GCH = 4000
NGC = V // GCH
VREGS_ROW = V // L
VREGS_GCH = GCH // L

INBIN_CAP = 3072
CAND_CAP = 32

BIG = 3.0e38
BIGI = 2**30


def _f1_of(xv):
    t = ((xv - LO) * INV_W1).astype(jnp.int32)
    return jnp.minimum(jnp.maximum(t, 0), NB1 - 1)


def _f2_of(xv, lo2):
    t = ((xv - lo2) * INV_W2).astype(jnp.int32)
    return jnp.minimum(jnp.maximum(t, 0), NB2 - 1)


def _body(x_hbm, g_hbm, out_hbm, xrow, hflat, gbuf0, gbuf1, tref,
          lst_x, lst_g, lst_i, cand_x, cand_g, cand_i, resbuf, sem0, sem1):
    wid = lax.axis_index("s") * 2 + lax.axis_index("c")
    iota = lax.iota(jnp.int32, L)
    fzero = jnp.zeros((L,), jnp.float32)
    izero = jnp.zeros((L,), jnp.int32)
    sil_mask = iota < len(SIL)
    sil_idx = izero
    for k, s in enumerate(SIL):
        sil_idx = jnp.where(iota == k, s, sil_idx)

    def zero_h(nwords):
        @plsc.parallel_loop(0, nwords // L, unroll=8)
        def _(i):
            hflat[pl.ds(i * L, L)] = fzero

    def bin_totals(nb):
        @plsc.parallel_loop(0, nb // L, unroll=2)
        def _(cb):
            acc = fzero
            for k in range(L):
                s = jnp.sum(hflat[pl.ds((cb * L + k) * L, L)])
                acc = jnp.where(iota == k, s, acc)
            tref[pl.ds(cb * L, L)] = acc

    def find_cross(nb, a0, p_c):
        def st(t, carry):
            acc, found, beta, a = carry
            cb = nb // L - 1 - t
            tv = tref[pl.ds(cb * L, L)]
            rv = lax.rev(tv, (0,))
            cs = plsc.cumsum(rv)
            m = (acc + cs) > p_c
            anyc = jnp.any(m)
            k = jnp.min(jnp.where(m, iota, L))
            ak = acc + jnp.sum(jnp.where(iota < k, rv, 0.0))
            betak = cb * L + (L - 1) - k
            use = anyc & jnp.logical_not(found)
            beta = jnp.where(use, betak, beta)
            a = jnp.where(use, ak, a)
            return (acc + jnp.sum(tv), found | anyc, beta, a)
        _, _, beta, a = lax.fori_loop(
            0, nb // L, st, (a0, False, jnp.int32(-1), jnp.float32(0.0)))
        return beta, a

    def row_body(j, resvec):
        row = wid * ROWS_PER_W + j
        pltpu.sync_copy(x_hbm.at[pl.ds(row * V, V)], xrow)
        plsc.addupdate_scatter(xrow, [sil_idx],
                               jnp.full((L,), -PEN_W, jnp.float32),
                               mask=sil_mask)
        zero_h(NB1 * L)

        @plsc.parallel_loop(0, VREGS_ROW, unroll=8)
        def _(i):
            xv = xrow[pl.ds(i * L, L)]
            ev = jnp.exp(xv)
            addr = _f1_of(xv) * L + iota
            plsc.addupdate_scatter(hflat, [addr], ev)
        bin_totals(NB1)

        def csum(cb, acc):
            return acc + tref[pl.ds(cb * L, L)]
        c_total = jnp.sum(lax.fori_loop(0, NB1 // L, csum, fzero))
        p_c = jnp.float32(TOPP) * c_total
        beta1, a1 = find_cross(NB1, jnp.float32(0.0), p_c)
        lo2 = jnp.float32(LO) + beta1.astype(jnp.float32) * jnp.float32(W1)

        def chunk_body(cbase, gbuf, carry):
            @plsc.parallel_loop(0, VREGS_GCH, unroll=4, carry=carry)
            def p4(i, car):
                bestv, besti, cntv = car
                xv = xrow[pl.ds(cbase + i * L, L)]
                gv = gbuf[pl.ds(i * L, L)]
                f1 = _f1_of(xv)
                scv = jnp.where(f1 > beta1, xv + gv, -BIG)
                idxv = iota + (cbase + i * L)
                upd = scv > bestv
                bestv = jnp.where(upd, scv, bestv)
                besti = jnp.where(upd, idxv, besti)
                inb = f1 == beta1
                n = plsc.all_reduce_population_count(inb)

                @pl.when(n[0] > 0)
                def _():
                    pos = jnp.minimum(
                        cntv + plsc.cumsum(inb.astype(jnp.int32)) - 1,
                        INBIN_CAP - 1)
                    plsc.store_scatter(lst_x, [pos], xv, mask=inb)
                    plsc.store_scatter(lst_g, [pos], gv, mask=inb)
                    plsc.store_scatter(lst_i, [pos], idxv, mask=inb)
                return (bestv, besti, cntv + n)
            return p4

        carry = (jnp.full((L,), -BIG, jnp.float32), izero, izero)
        gbufs = (gbuf0, gbuf1)
        sems = (sem0, sem1)
        gbase = row * V
        desc = pltpu.async_copy(g_hbm.at[pl.ds(gbase, GCH)], gbuf0, sem0)
        for c in range(NGC):
            nxt = None
            if c + 1 < NGC:
                nxt = pltpu.async_copy(
                    g_hbm.at[pl.ds(gbase + (c + 1) * GCH, GCH)],
                    gbufs[(c + 1) % 2], sems[(c + 1) % 2])
            desc.wait()
            carry = chunk_body(c * GCH, gbufs[c % 2], carry)
            desc = nxt
        bestv, besti, cntv = carry
        ninb = cntv[0]
        nvr = (ninb + (L - 1)) >> 4

        zero_h(NB2 * L)

        def sh(i, c):
            xv = lst_x[pl.ds(i * L, L)]
            ev = jnp.exp(xv)
            valid = (iota + i * L) < ninb
            addr = _f2_of(xv, lo2) * L + iota
            plsc.addupdate_scatter(hflat, [addr], ev, mask=valid)
            return c
        lax.fori_loop(0, nvr, sh, 0)
        bin_totals(NB2)
        beta2, a2 = find_cross(NB2, a1, p_c)

        def pb(i, car):
            bestv, besti, cntc = car
            xv = lst_x[pl.ds(i * L, L)]
            gv = lst_g[pl.ds(i * L, L)]
            iv = lst_i[pl.ds(i * L, L)]
            valid = (iota + i * L) < ninb
            f2 = _f2_of(xv, lo2)
            scv = jnp.where(valid & (f2 > beta2), xv + gv, -BIG)
            upd = (scv > bestv) | ((scv == bestv) & (iv < besti))
            bestv = jnp.where(upd, scv, bestv)
            besti = jnp.where(upd, iv, besti)
            candm = valid & (f2 == beta2)
            n = plsc.all_reduce_population_count(candm)

            @pl.when(n[0] > 0)
            def _():
                pos = jnp.minimum(
                    cntc + plsc.cumsum(candm.astype(jnp.int32)) - 1,
                    CAND_CAP - 1)
                plsc.store_scatter(cand_x, [pos], xv, mask=candm)
                plsc.store_scatter(cand_g, [pos], gv, mask=candm)
                plsc.store_scatter(cand_i, [pos], iv, mask=candm)
            return (bestv, besti, cntc + n)
        bestv, besti, cntc = lax.fori_loop(0, nvr, pb, (bestv, besti, izero))

        ma = jnp.max(bestv)
        ia = jnp.min(jnp.where(bestv == ma, besti, BIGI))
        ncand = cntc[0]

        cx0 = cand_x[pl.ds(0, L)]
        cx1 = cand_x[pl.ds(L, L)]
        cg0 = cand_g[pl.ds(0, L)]
        cg1 = cand_g[pl.ds(L, L)]
        ci0 = cand_i[pl.ds(0, L)]
        ci1 = cand_i[pl.ds(L, L)]
        ce0 = jnp.exp(cx0)
        ce1 = jnp.exp(cx1)
        sg0, sg1 = fzero, fzero
        eq0, eq1 = izero, izero
        for jj in range(CAND_CAP):
            validj = ncand > jj
            if jj < L:
                xj, ij, ej = cx0[jj], ci0[jj], ce0[jj]
            else:
                xj, ij, ej = cx1[jj - L], ci1[jj - L], ce1[jj - L]
            sg0 = sg0 + jnp.where(validj & (xj > cx0), ej, 0.0)
            sg1 = sg1 + jnp.where(validj & (xj > cx1), ej, 0.0)
            eq0 = eq0 + jnp.where(validj & (xj == cx0) & (ij <= ci0), 1, 0)
            eq1 = eq1 + jnp.where(validj & (xj == cx1) & (ij <= ci1), 1, 0)
        lv0 = iota < ncand
        lv1 = (iota + L) < ncand
        cv0 = a2 + sg0 + ce0 * eq0.astype(jnp.float32)
        cv1 = a2 + sg1 + ce1 * eq1.astype(jnp.float32)
        ck0 = lv0 & ((cv0 - ce0) <= p_c)
        ck1 = lv1 & ((cv1 - ce1) <= p_c)
        cs0 = jnp.where(ck0, cx0 + cg0, -BIG)
        cs1 = jnp.where(ck1, cx1 + cg1, -BIG)
        mc = jnp.maximum(jnp.max(cs0), jnp.max(cs1))
        ic = jnp.minimum(jnp.min(jnp.where(cs0 == mc, ci0, BIGI)),
                         jnp.min(jnp.where(cs1 == mc, ci1, BIGI)))
        use_c = (mc > ma) | ((mc == ma) & (ic < ia))
        res = jnp.where(use_c, ic, ia)
        return jnp.where(iota == j, res, resvec)

    resvec = lax.fori_loop(0, ROWS_PER_W, row_body, jnp.zeros((L,), jnp.int32))
    resbuf[pl.ds(0, L)] = resvec
    pltpu.sync_copy(resbuf, out_hbm.at[wid])


def _gumbel_const():
    skey = jax.random.fold_in(jax.random.key(0), 1)
    return jax.random.gumbel(skey, (B * V,), jnp.float32)


@functools.cache
def _sc_call():
    mesh = plsc.VectorSubcoreMesh(core_axis_name="c", subcore_axis_name="s",
                                  num_cores=2, num_subcores=16)
    return pl.kernel(
        _body,
        out_type=jax.ShapeDtypeStruct((NW, L), jnp.int32),
        mesh=mesh,
        compiler_params=pltpu.CompilerParams(use_tc_tiling_on_sc=False,
                                             needs_layout_passes=False),
        scratch_types=[
            pltpu.VMEM((V,), jnp.float32),
            pltpu.VMEM((NB1 * L,), jnp.float32),
            pltpu.VMEM((GCH,), jnp.float32),
            pltpu.VMEM((GCH,), jnp.float32),
            pltpu.VMEM((NB1,), jnp.float32),
            pltpu.VMEM((INBIN_CAP,), jnp.float32),
            pltpu.VMEM((INBIN_CAP,), jnp.float32),
            pltpu.VMEM((INBIN_CAP,), jnp.int32),
            pltpu.VMEM((CAND_CAP,), jnp.float32),
            pltpu.VMEM((CAND_CAP,), jnp.float32),
            pltpu.VMEM((CAND_CAP,), jnp.int32),
            pltpu.VMEM((L,), jnp.int32),
            pltpu.SemaphoreType.DMA,
            pltpu.SemaphoreType.DMA,
        ],
    )


def kernel(logits):
    assert logits.shape == (B, V) and logits.dtype == jnp.float32
    staging = _sc_call()(logits.reshape(B * V), _gumbel_const())
    return staging[:, :ROWS_PER_W].reshape(B, 1).astype(jnp.int32)


# speedup vs baseline: 1.8403x; 1.0401x over previous
import functools

import jax
import jax.numpy as jnp
from jax import lax
from jax.experimental import pallas as pl
from jax.experimental.pallas import tpu as pltpu
from jax.experimental.pallas import tpu_sc as plsc

B = 128
V = 100000
TOPP = 0.95
PEN_W = 10.0
SIL = (1049, 127, 1880, 1492, 972, 1031, 395, 2029, 581, 175, 1926, 407, 1316)

L = 16
NW = 32
ROWS_PER_W = B // NW

NB1 = 512
NB2 = 512
LO = -26.0
W1 = 32.0 / NB1
W2 = W1 / NB2
INV_W1 = 1.0 / W1
INV_W2 = 1.0 / W2

GCH = 4000
NGC = V // GCH
VREGS_ROW = V // L
VREGS_GCH = GCH // L

INBIN_CAP = 3072
CAND_CAP = 32

BIG = 3.0e38
BIGI = 2**30


def _f1_of(xv):
    t = ((xv - LO) * INV_W1).astype(jnp.int32)
    return jnp.minimum(jnp.maximum(t, 0), NB1 - 1)


def _f2_of(xv, lo2):
    t = ((xv - lo2) * INV_W2).astype(jnp.int32)
    return jnp.minimum(jnp.maximum(t, 0), NB2 - 1)


def _sil_vec(iota):
    sil = jnp.zeros((L,), jnp.int32)
    for k, s in enumerate(SIL):
        sil = jnp.where(iota == k, s, sil)
    return sil, iota < len(SIL)


def _mk_helpers(hflat, tref, iota, fzero):
    def zero_h(nwords):
        @plsc.parallel_loop(0, nwords // L, unroll=8)
        def _(i):
            hflat[pl.ds(i * L, L)] = fzero

    def bin_totals(nb):
        @plsc.parallel_loop(0, nb // L, unroll=2)
        def _(cb):
            acc = fzero
            for k in range(L):
                s = jnp.sum(hflat[pl.ds((cb * L + k) * L, L)])
                acc = jnp.where(iota == k, s, acc)
            tref[pl.ds(cb * L, L)] = acc

    def find_cross(nb, a0, p_c):
        def st(t, carry):
            acc, found, beta, a = carry
            cb = nb // L - 1 - t
            tv = tref[pl.ds(cb * L, L)]
            rv = lax.rev(tv, (0,))
            cs = plsc.cumsum(rv)
            m = (acc + cs) > p_c
            anyc = jnp.any(m)
            k = jnp.min(jnp.where(m, iota, L))
            ak = acc + jnp.sum(jnp.where(iota < k, rv, 0.0))
            betak = cb * L + (L - 1) - k
            use = anyc & jnp.logical_not(found)
            beta = jnp.where(use, betak, beta)
            a = jnp.where(use, ak, a)
            return (acc + jnp.sum(tv), found | anyc, beta, a)
        _, _, beta, a = lax.fori_loop(
            0, nb // L, st, (a0, False, jnp.int32(-1), jnp.float32(0.0)))
        return beta, a

    return zero_h, bin_totals, find_cross


def _body_a(x_hbm, b1_hbm, a1_hbm, pc_hbm, xrow, hflat, tref,
            rb_i, rb_a, rb_p):
    wid = lax.axis_index("s") * 2 + lax.axis_index("c")
    iota = lax.iota(jnp.int32, L)
    fzero = jnp.zeros((L,), jnp.float32)
    sil_idx, sil_mask = _sil_vec(iota)
    zero_h, bin_totals, find_cross = _mk_helpers(hflat, tref, iota, fzero)

    def row_body(j, carry):
        b1acc, a1acc, pcacc = carry
        row = wid * ROWS_PER_W + j
        pltpu.sync_copy(x_hbm.at[pl.ds(row * V, V)], xrow)
        plsc.addupdate_scatter(xrow, [sil_idx],
                               jnp.full((L,), -PEN_W, jnp.float32),
                               mask=sil_mask)
        zero_h(NB1 * L)

        @plsc.parallel_loop(0, VREGS_ROW, unroll=8)
        def _(i):
            xv = xrow[pl.ds(i * L, L)]
            ev = jnp.exp(xv)
            addr = _f1_of(xv) * L + iota
            plsc.addupdate_scatter(hflat, [addr], ev)
        bin_totals(NB1)

        def csum(cb, acc):
            return acc + tref[pl.ds(cb * L, L)]
        c_total = jnp.sum(lax.fori_loop(0, NB1 // L, csum, fzero))
        p_c = jnp.float32(TOPP) * c_total
        beta1, a1 = find_cross(NB1, jnp.float32(0.0), p_c)
        return (jnp.where(iota == j, beta1, b1acc),
                jnp.where(iota == j, a1, a1acc),
                jnp.where(iota == j, p_c, pcacc))

    izero = jnp.zeros((L,), jnp.int32)
    b1v, a1v, pcv = lax.fori_loop(0, ROWS_PER_W, row_body,
                                  (izero, fzero, fzero))
    rb_i[pl.ds(0, L)] = b1v
    rb_a[pl.ds(0, L)] = a1v
    rb_p[pl.ds(0, L)] = pcv
    pltpu.sync_copy(rb_i, b1_hbm.at[wid])
    pltpu.sync_copy(rb_a, a1_hbm.at[wid])
    pltpu.sync_copy(rb_p, pc_hbm.at[wid])


def _body_b(x_hbm, g_hbm, b1_hbm, a1_hbm, pc_hbm, out_hbm, xrow, hflat,
            gbuf0, gbuf1, tref, lst_x, lst_g, lst_i, cand_x, cand_g, cand_i,
            resbuf, pb_i, pb_a, pb_p, sem0, sem1):
    wid = lax.axis_index("s") * 2 + lax.axis_index("c")
    iota = lax.iota(jnp.int32, L)
    fzero = jnp.zeros((L,), jnp.float32)
    izero = jnp.zeros((L,), jnp.int32)
    sil_idx, sil_mask = _sil_vec(iota)
    zero_h, bin_totals, find_cross = _mk_helpers(hflat, tref, iota, fzero)

    pltpu.sync_copy(b1_hbm.at[wid], pb_i)
    pltpu.sync_copy(a1_hbm.at[wid], pb_a)
    pltpu.sync_copy(pc_hbm.at[wid], pb_p)
    b1v = pb_i[pl.ds(0, L)]
    a1v = pb_a[pl.ds(0, L)]
    pcv = pb_p[pl.ds(0, L)]

    def row_body(j, resvec):
        row = wid * ROWS_PER_W + j
        jm = iota == j
        beta1 = jnp.max(jnp.where(jm, b1v, -1))
        a1 = jnp.sum(jnp.where(jm, a1v, 0.0))
        p_c = jnp.sum(jnp.where(jm, pcv, 0.0))
        lo2 = jnp.float32(LO) + beta1.astype(jnp.float32) * jnp.float32(W1)

        pltpu.sync_copy(x_hbm.at[pl.ds(row * V, V)], xrow)
        plsc.addupdate_scatter(xrow, [sil_idx],
                               jnp.full((L,), -PEN_W, jnp.float32),
                               mask=sil_mask)

        def chunk_body(cbase, gbuf, carry):
            @plsc.parallel_loop(0, VREGS_GCH, unroll=4, carry=carry)
            def p4(i, car):
                bestv, besti, cntv = car
                xv = xrow[pl.ds(cbase + i * L, L)]
                gv = gbuf[pl.ds(i * L, L)]
                f1 = _f1_of(xv)
                scv = jnp.where(f1 > beta1, xv + gv, -BIG)
                idxv = iota + (cbase + i * L)
                upd = scv > bestv
                bestv = jnp.where(upd, scv, bestv)
                besti = jnp.where(upd, idxv, besti)
                inb = f1 == beta1
                n = plsc.all_reduce_population_count(inb)

                @pl.when(n[0] > 0)
                def _():
                    pos = jnp.minimum(
                        cntv + plsc.cumsum(inb.astype(jnp.int32)) - 1,
                        INBIN_CAP - 1)
                    plsc.store_scatter(lst_x, [pos], xv, mask=inb)
                    plsc.store_scatter(lst_g, [pos], gv, mask=inb)
                    plsc.store_scatter(lst_i, [pos], idxv, mask=inb)
                return (bestv, besti, cntv + n)
            return p4

        carry = (jnp.full((L,), -BIG, jnp.float32), izero, izero)
        gbufs = (gbuf0, gbuf1)
        sems = (sem0, sem1)
        gbase = row * V
        desc = pltpu.async_copy(g_hbm.at[pl.ds(gbase, GCH)], gbuf0, sem0)
        for c in range(NGC):
            nxt = None
            if c + 1 < NGC:
                nxt = pltpu.async_copy(
                    g_hbm.at[pl.ds(gbase + (c + 1) * GCH, GCH)],
                    gbufs[(c + 1) % 2], sems[(c + 1) % 2])
            desc.wait()
            carry = chunk_body(c * GCH, gbufs[c % 2], carry)
            desc = nxt
        bestv, besti, cntv = carry
        ninb = cntv[0]
        nvr = (ninb + (L - 1)) >> 4

        zero_h(NB2 * L)

        def sh(i, c):
            xv = lst_x[pl.ds(i * L, L)]
            ev = jnp.exp(xv)
            valid = (iota + i * L) < ninb
            addr = _f2_of(xv, lo2) * L + iota
            plsc.addupdate_scatter(hflat, [addr], ev, mask=valid)
            return c
        lax.fori_loop(0, nvr, sh, 0)
        bin_totals(NB2)
        beta2, a2 = find_cross(NB2, a1, p_c)

        def pbody(i, car):
            bestv, besti, cntc = car
            xv = lst_x[pl.ds(i * L, L)]
            gv = lst_g[pl.ds(i * L, L)]
            iv = lst_i[pl.ds(i * L, L)]
            valid = (iota + i * L) < ninb
            f2 = _f2_of(xv, lo2)
            scv = jnp.where(valid & (f2 > beta2), xv + gv, -BIG)
            upd = (scv > bestv) | ((scv == bestv) & (iv < besti))
            bestv = jnp.where(upd, scv, bestv)
            besti = jnp.where(upd, iv, besti)
            candm = valid & (f2 == beta2)
            n = plsc.all_reduce_population_count(candm)

            @pl.when(n[0] > 0)
            def _():
                pos = jnp.minimum(
                    cntc + plsc.cumsum(candm.astype(jnp.int32)) - 1,
                    CAND_CAP - 1)
                plsc.store_scatter(cand_x, [pos], xv, mask=candm)
                plsc.store_scatter(cand_g, [pos], gv, mask=candm)
                plsc.store_scatter(cand_i, [pos], iv, mask=candm)
            return (bestv, besti, cntc + n)
        bestv, besti, cntc = lax.fori_loop(0, nvr, pbody,
                                           (bestv, besti, izero))

        ma = jnp.max(bestv)
        ia = jnp.min(jnp.where(bestv == ma, besti, BIGI))
        ncand = cntc[0]

        cx0 = cand_x[pl.ds(0, L)]
        cx1 = cand_x[pl.ds(L, L)]
        cg0 = cand_g[pl.ds(0, L)]
        cg1 = cand_g[pl.ds(L, L)]
        ci0 = cand_i[pl.ds(0, L)]
        ci1 = cand_i[pl.ds(L, L)]
        ce0 = jnp.exp(cx0)
        ce1 = jnp.exp(cx1)
        sg0, sg1 = fzero, fzero
        eq0, eq1 = izero, izero
        for jj in range(CAND_CAP):
            validj = ncand > jj
            if jj < L:
                xj, ij, ej = cx0[jj], ci0[jj], ce0[jj]
            else:
                xj, ij, ej = cx1[jj - L], ci1[jj - L], ce1[jj - L]
            sg0 = sg0 + jnp.where(validj & (xj > cx0), ej, 0.0)
            sg1 = sg1 + jnp.where(validj & (xj > cx1), ej, 0.0)
            eq0 = eq0 + jnp.where(validj & (xj == cx0) & (ij <= ci0), 1, 0)
            eq1 = eq1 + jnp.where(validj & (xj == cx1) & (ij <= ci1), 1, 0)
        lv0 = iota < ncand
        lv1 = (iota + L) < ncand
        cv0 = a2 + sg0 + ce0 * eq0.astype(jnp.float32)
        cv1 = a2 + sg1 + ce1 * eq1.astype(jnp.float32)
        ck0 = lv0 & ((cv0 - ce0) <= p_c)
        ck1 = lv1 & ((cv1 - ce1) <= p_c)
        cs0 = jnp.where(ck0, cx0 + cg0, -BIG)
        cs1 = jnp.where(ck1, cx1 + cg1, -BIG)
        mc = jnp.maximum(jnp.max(cs0), jnp.max(cs1))
        ic = jnp.minimum(jnp.min(jnp.where(cs0 == mc, ci0, BIGI)),
                         jnp.min(jnp.where(cs1 == mc, ci1, BIGI)))
        use_c = (mc > ma) | ((mc == ma) & (ic < ia))
        res = jnp.where(use_c, ic, ia)
        return jnp.where(jm, res, resvec)

    resvec = lax.fori_loop(0, ROWS_PER_W, row_body, jnp.zeros((L,), jnp.int32))
    resbuf[pl.ds(0, L)] = resvec
    pltpu.sync_copy(resbuf, out_hbm.at[wid])


def _gumbel_const():
    skey = jax.random.fold_in(jax.random.key(0), 1)
    return jax.random.gumbel(skey, (B * V,), jnp.float32)


def _mesh():
    return plsc.VectorSubcoreMesh(core_axis_name="c", subcore_axis_name="s",
                                  num_cores=2, num_subcores=16)


_CP = dict(use_tc_tiling_on_sc=False, needs_layout_passes=False)


@functools.cache
def _sc_call_a():
    return pl.kernel(
        _body_a,
        out_type=(jax.ShapeDtypeStruct((NW, L), jnp.int32),
                  jax.ShapeDtypeStruct((NW, L), jnp.float32),
                  jax.ShapeDtypeStruct((NW, L), jnp.float32)),
        mesh=_mesh(),
        compiler_params=pltpu.CompilerParams(**_CP),
        scratch_types=[
            pltpu.VMEM((V,), jnp.float32),
            pltpu.VMEM((NB1 * L,), jnp.float32),
            pltpu.VMEM((NB1,), jnp.float32),
            pltpu.VMEM((L,), jnp.int32),
            pltpu.VMEM((L,), jnp.float32),
            pltpu.VMEM((L,), jnp.float32),
        ],
    )


@functools.cache
def _sc_call_b():
    return pl.kernel(
        _body_b,
        out_type=jax.ShapeDtypeStruct((NW, L), jnp.int32),
        mesh=_mesh(),
        compiler_params=pltpu.CompilerParams(**_CP),
        scratch_types=[
            pltpu.VMEM((V,), jnp.float32),
            pltpu.VMEM((NB2 * L,), jnp.float32),
            pltpu.VMEM((GCH,), jnp.float32),
            pltpu.VMEM((GCH,), jnp.float32),
            pltpu.VMEM((NB2,), jnp.float32),
            pltpu.VMEM((INBIN_CAP,), jnp.float32),
            pltpu.VMEM((INBIN_CAP,), jnp.float32),
            pltpu.VMEM((INBIN_CAP,), jnp.int32),
            pltpu.VMEM((CAND_CAP,), jnp.float32),
            pltpu.VMEM((CAND_CAP,), jnp.float32),
            pltpu.VMEM((CAND_CAP,), jnp.int32),
            pltpu.VMEM((L,), jnp.int32),
            pltpu.VMEM((L,), jnp.int32),
            pltpu.VMEM((L,), jnp.float32),
            pltpu.VMEM((L,), jnp.float32),
            pltpu.SemaphoreType.DMA,
            pltpu.SemaphoreType.DMA,
        ],
    )


def kernel(logits):
    assert logits.shape == (B, V) and logits.dtype == jnp.float32
    xflat = logits.reshape(B * V)
    b1, a1v, pcv = _sc_call_a()(xflat)
    staging = _sc_call_b()(xflat, _gumbel_const(), b1, a1v, pcv)
    return staging[:, :ROWS_PER_W].reshape(B, 1).astype(jnp.int32)
